# speedup vs baseline: 13.3588x; 13.3588x over previous
import functools

import jax
import jax.numpy as jnp
from jax import lax
from jax.experimental import pallas as pl
from jax.experimental.pallas import tpu as pltpu
from jax.experimental.pallas import tpu_sc as plsc


def _gather_sum(x3, table, B, CH, CL, D):
  info = plsc.get_sparse_core_info()
  NC, NS = info.num_cores, info.num_subcores
  NW = NC * NS
  assert B % NW == 0
  SPW = B // NW
  assert SPW % 2 == 0
  NV = D // 16

  mesh = plsc.VectorSubcoreMesh(core_axis_name="c", subcore_axis_name="s")

  @functools.partial(
      pl.kernel,
      mesh=mesh,
      out_type=jax.ShapeDtypeStruct((B, D), jnp.float32),
      scratch_types=[
          pltpu.VMEM((SPW, CH, CL), jnp.int32),
          pltpu.VMEM((2, CH, CL, D), jnp.float32),
          pltpu.VMEM((SPW, D), jnp.float32),
          pltpu.SemaphoreType.DMA,
          pltpu.SemaphoreType.DMA,
      ],
  )
  def k(x_hbm, table_hbm, out_hbm, idx_v, rows_v, acc_v, sem0, sem1):
    cid = lax.axis_index("c")
    sid = lax.axis_index("s")
    wid = sid * NC + cid
    base = wid * SPW

    pltpu.sync_copy(x_hbm.at[pl.ds(base, SPW)], idx_v)
    sems = (sem0, sem1)

    def fire(si, slot):
      for ch in range(CH):
        pltpu.make_async_copy(
            table_hbm.at[idx_v.at[si, ch]],
            rows_v.at[slot, ch],
            sems[slot],
        ).start()

    def drain(slot):
      for ch in range(CH):
        pltpu.make_async_copy(
            table_hbm.at[idx_v.at[0, ch]],
            rows_v.at[slot, ch],
            sems[slot],
        ).wait()

    def accumulate(slot):
      zero = jnp.zeros((16,), jnp.float32)
      acc0 = (zero,) * (NV * CH)

      def body(r, acc):
        out = []
        for ch in range(CH):
          for j in range(NV):
            out.append(
                acc[ch * NV + j] + rows_v[slot, ch, r, pl.ds(j * 16, 16)])
        return tuple(out)

      acc = lax.fori_loop(0, CL, body, acc0)
      return tuple(
          functools.reduce(lambda a, b: a + b,
                           [acc[ch * NV + j] for ch in range(CH)])
          for j in range(NV))

    fire(0, 0)
    fire(1, 1)

    def samp_body(i2, carry):
      for slot in range(2):
        si = i2 * 2 + slot
        drain(slot)
        acc = accumulate(slot)

        @pl.when(si + 2 < SPW)
        def _():
          fire(si + 2, slot)

        for j in range(NV):
          acc_v[si, pl.ds(j * 16, 16)] = acc[j]
      return carry

    lax.fori_loop(0, SPW // 2, samp_body, 0)
    pltpu.sync_copy(acc_v, out_hbm.at[pl.ds(base, SPW)])

  return k(x3, table)


def _mlp_body(rep_ref, len_ref, w1t_ref, b1_ref, w2t_ref, b2_ref, out_ref):
  rep = rep_ref[...] / len_ref[...]
  h = jnp.dot(rep, w1t_ref[...], preferred_element_type=jnp.float32)
  h = jnp.maximum(h + b1_ref[...], 0.0)
  out = jnp.dot(h, w2t_ref[...], preferred_element_type=jnp.float32)
  out_ref[...] = out + b2_ref[...]


def kernel(x, lengths, table, W1, b1, W2, b2):
  B, L = x.shape
  D = table.shape[1]
  H = W1.shape[0]
  O = W2.shape[0]

  CH = -(-L // 128)
  assert L % CH == 0
  CL = L // CH
  x3 = x.reshape(B, CH, CL)

  rep_sum = _gather_sum(x3, table, B, CH, CL, D)

  lens = lengths.astype(jnp.float32).reshape(B, 1)
  logits = pl.pallas_call(
      _mlp_body,
      out_shape=jax.ShapeDtypeStruct((B, O), jnp.float32),
  )(rep_sum, lens, W1.T, b1.reshape(1, H), W2.T, b2.reshape(1, O))
  return logits
